# baseline (speedup 1.0000x reference)
import functools

import jax
import jax.numpy as jnp
from jax import lax
from jax.experimental import pallas as pl
from jax.experimental.pallas import tpu as pltpu
from jax.experimental.pallas import tpu_sc as plsc

N = 10000
E = 320000
D = 128
R = 8
NC = 2
NS = 16
L = 16
EB = 80
GS = 4

CNT_PAD = 81920
E_PER_TILE_ALL = E // NS
E_PER_TILE_HALF = E // (NC * NS)

_mesh = plsc.VectorSubcoreMesh(
    core_axis_name="c", subcore_axis_name="s", num_cores=NC, num_subcores=NS)


def _zero_fill(ref, nwords):
    z = jnp.zeros((L,), dtype=ref.dtype)

    def body(i, _):
        ref[pl.ds(i * L, L)] = z
        return 0

    lax.fori_loop(0, nwords // L, body, 0)


def _grouped(n_blocks, group_fn):
    nfull = n_blocks // GS
    rem = n_blocks - nfull * GS

    def outer(go, _):
        group_fn(go * GS, GS)
        return 0

    lax.fori_loop(0, nfull, outer, 0)
    if rem:
        group_fn(nfull * GS, rem)


@functools.partial(
    pl.kernel,
    out_type=(
        jax.ShapeDtypeStruct((E,), jnp.float32),
        jax.ShapeDtypeStruct((E,), jnp.int32),
    ),
    mesh=_mesh,
    scratch_types=(
        pltpu.VMEM((CNT_PAD // NS,), jnp.float32),
        pltpu.VMEM((GS, EB), jnp.int32),
        pltpu.VMEM((GS, EB), jnp.int32),
        pltpu.VMEM((GS, EB), jnp.int32),
        pltpu.VMEM((GS, EB), jnp.int32),
        pltpu.VMEM((GS, EB), jnp.int32),
        pltpu.VMEM((GS, EB), jnp.float32),
        pltpu.VMEM((GS, EB), jnp.float32),
        pltpu.VMEM((EB,), jnp.float32),
        pltpu.VMEM_SHARED((CNT_PAD,), jnp.float32),
        (pltpu.SemaphoreType.DMA,) * GS,
        (pltpu.SemaphoreType.DMA,) * GS,
        (pltpu.SemaphoreType.DMA,) * GS,
    ),
)
def _norm_kernel(src_hbm, dst_hbm, et_hbm, norm_out, gidx_out,
                 zb, db, eb, sb, cb, gb, cntb, nb, ones, cnt_s,
                 sem_l, sem_cg, sem_st):
    c = lax.axis_index("c")
    s = lax.axis_index("s")

    _zero_fill(zb, CNT_PAD // NS)
    pltpu.sync_copy(zb, cnt_s.at[pl.ds(s * (CNT_PAD // NS), CNT_PAD // NS)])
    for k in range(EB // L):
        ones[pl.ds(k * L, L)] = jnp.ones((L,), jnp.float32)
    plsc.subcore_barrier()

    bbase = s * E_PER_TILE_ALL

    def count_group(blk0, count):
        base0 = bbase + blk0 * EB
        hs = []
        for j in range(count):
            base = base0 + j * EB
            hs.append((
                pltpu.async_copy(
                    dst_hbm.at[pl.ds(base, EB)], db.at[j], sem_l[j]),
                pltpu.async_copy(
                    et_hbm.at[pl.ds(base, EB)], eb.at[j], sem_l[j]),
            ))
        sh = []
        for j in range(count):
            for h in hs[j]:
                h.wait()
            for k in range(EB // L):
                ds = pl.ds(k * L, L)
                cb[j, ds] = db[j, ds] * R + eb[j, ds]
            sh.append(pltpu.async_copy(
                ones, cnt_s.at[cb.at[j]], sem_cg[j], add=True))
        for h in sh:
            h.wait()

    _grouped(E_PER_TILE_ALL // EB, count_group)
    plsc.subcore_barrier()

    cbase = c * (E // NC) + s * E_PER_TILE_HALF

    def norm_group(blk0, count):
        base0 = cbase + blk0 * EB
        hs = []
        for j in range(count):
            base = base0 + j * EB
            hs.append((
                pltpu.async_copy(
                    dst_hbm.at[pl.ds(base, EB)], db.at[j], sem_l[j]),
                pltpu.async_copy(
                    et_hbm.at[pl.ds(base, EB)], eb.at[j], sem_l[j]),
                pltpu.async_copy(
                    src_hbm.at[pl.ds(base, EB)], sb.at[j], sem_l[j]),
            ))
        cgh = []
        for j in range(count):
            for h in hs[j]:
                h.wait()
            for k in range(EB // L):
                ds = pl.ds(k * L, L)
                cb[j, ds] = db[j, ds] * R + eb[j, ds]
                gb[j, ds] = (eb[j, ds] + 1) * N + sb[j, ds]
            cgh.append(pltpu.async_copy(
                cnt_s.at[cb.at[j]], cntb.at[j], sem_cg[j]))
        sth = []
        for j in range(count):
            base = base0 + j * EB
            cgh[j].wait()
            for k in range(EB // L):
                ds = pl.ds(k * L, L)
                nb[j, ds] = 1.0 / jnp.maximum(cntb[j, ds], 1.0)
            sth.append(pltpu.async_copy(
                nb.at[j], norm_out.at[pl.ds(base, EB)], sem_st[j]))
            sth.append(pltpu.async_copy(
                gb.at[j], gidx_out.at[pl.ds(base, EB)], sem_st[j]))
        for h in sth:
            h.wait()

    _grouped(E_PER_TILE_HALF // EB, norm_group)


NP = 10240
N_PER_TILE = NP // NS


@functools.partial(
    pl.kernel,
    out_type=jax.ShapeDtypeStruct((NC, NP, D), jnp.float32),
    mesh=_mesh,
    scratch_types=(
        pltpu.VMEM((GS, EB), jnp.int32),
        pltpu.VMEM((GS, EB), jnp.int32),
        pltpu.VMEM((GS, EB + L), jnp.float32),
        pltpu.VMEM((GS, EB, D), jnp.float32),
        pltpu.VMEM_SHARED((NP, D), jnp.float32),
        (pltpu.SemaphoreType.DMA,) * GS,
        (pltpu.SemaphoreType.DMA,) * GS,
        (pltpu.SemaphoreType.DMA,) * GS,
    ),
)
def _agg_kernel(table_hbm, dst_hbm, gidx_hbm, norm_hbm, out,
                gb, db, nb, rows, acc_s, sem_l, sem_g, sem_s):
    c = lax.axis_index("c")
    s = lax.axis_index("s")
    ebase = c * (E // NC) + s * E_PER_TILE_HALF

    def zbody(i, _):
        for k in range(D // L):
            rows[0, i, pl.ds(k * L, L)] = jnp.zeros((L,), jnp.float32)
        return 0

    lax.fori_loop(0, EB, zbody, 0)
    for i in range(N_PER_TILE // EB):
        pltpu.sync_copy(
            rows.at[0], acc_s.at[pl.ds(s * N_PER_TILE + i * EB, EB), :])
    plsc.subcore_barrier()

    def edge_group(blk0, count):
        base0 = ebase + blk0 * EB
        hs = []
        for j in range(count):
            base = base0 + j * EB
            hs.append((
                pltpu.async_copy(
                    gidx_hbm.at[pl.ds(base, EB)], gb.at[j], sem_l[j]),
                pltpu.async_copy(
                    dst_hbm.at[pl.ds(base, EB)], db.at[j], sem_l[j]),
                pltpu.async_copy(
                    norm_hbm.at[pl.ds(base, EB)], nb.at[j, pl.ds(0, EB)],
                    sem_l[j]),
            ))
        gh = []
        for j in range(count):
            for h in hs[j]:
                h.wait()
            gh.append(pltpu.async_copy(
                table_hbm.at[gb.at[j]], rows.at[j], sem_g[j]))
        sh = []
        for j in range(count):
            gh[j].wait()

            def scale_chunk(m, _, j=j):
                nv16 = nb[j, pl.ds(m * L, L)]
                b0 = m * L
                for i in range(L):
                    nv = nv16[i]
                    for k in range(D // L):
                        ds = pl.ds(k * L, L)
                        rows[j, b0 + i, ds] = rows[j, b0 + i, ds] * nv
                return 0

            lax.fori_loop(0, EB // L, scale_chunk, 0)
            sh.append(pltpu.async_copy(
                rows.at[j], acc_s.at[db.at[j]], sem_s[j], add=True))
        for h in sh:
            h.wait()

    _grouped(E_PER_TILE_HALF // EB, edge_group)
    plsc.subcore_barrier()

    for i in range(N_PER_TILE // EB):
        rs = pl.ds(s * N_PER_TILE + i * EB, EB)
        pltpu.sync_copy(acc_s.at[rs, :], out.at[c, rs, :])


BN = 1000


def _transform(xin, wcat):
    nb = N // BN

    def body(x_ref, w_ref, o_ref):
        o_ref[...] = jnp.dot(x_ref[...], w_ref[0],
                             preferred_element_type=jnp.float32)

    return pl.pallas_call(
        body,
        grid=(R + 1, nb),
        in_specs=[
            pl.BlockSpec((BN, D), lambda g, n: (n, 0)),
            pl.BlockSpec((1, D, D), lambda g, n: (g, 0, 0)),
        ],
        out_specs=pl.BlockSpec((BN, D), lambda g, n: (g * (N // BN) + n, 0)),
        out_shape=jax.ShapeDtypeStruct(((R + 1) * N, D), jnp.float32),
    )(xin, wcat)


def _transform_fused(partials, table, bias, wcat):
    nb = N // BN

    def body(p_ref, t_ref, b_ref, w_ref, o_ref):
        y = jnp.maximum(p_ref[0] + p_ref[1] + t_ref[...] + b_ref[0], 0.0)
        o_ref[...] = jnp.dot(y, w_ref[0], preferred_element_type=jnp.float32)

    return pl.pallas_call(
        body,
        grid=(R + 1, nb),
        in_specs=[
            pl.BlockSpec((NC, BN, D), lambda g, n: (0, n, 0)),
            pl.BlockSpec((BN, D), lambda g, n: (n, 0)),
            pl.BlockSpec((1, D), lambda g, n: (0, 0)),
            pl.BlockSpec((1, D, D), lambda g, n: (g, 0, 0)),
        ],
        out_specs=pl.BlockSpec((BN, D), lambda g, n: (g * (N // BN) + n, 0)),
        out_shape=jax.ShapeDtypeStruct(((R + 1) * N, D), jnp.float32),
    )(partials, table, bias, wcat)


def _combine(partials, table, bias, relu):
    def body(p_ref, t_ref, b_ref, o_ref):
        v = p_ref[0] + p_ref[1] + t_ref[...] + b_ref[0]
        if relu:
            v = jnp.maximum(v, 0.0)
        o_ref[...] = v

    return pl.pallas_call(
        body,
        grid=(N // BN,),
        in_specs=[
            pl.BlockSpec((NC, BN, D), lambda n: (0, n, 0)),
            pl.BlockSpec((BN, D), lambda n: (n, 0)),
            pl.BlockSpec((1, D), lambda n: (0, 0)),
        ],
        out_specs=pl.BlockSpec((BN, D), lambda n: (n, 0)),
        out_shape=jax.ShapeDtypeStruct((N, D), jnp.float32),
    )(partials, table, bias)


def kernel(x, edge_index, edge_type, Wr1, Wroot1, b1, Wr2, Wroot2, b2):
    src = edge_index[0]
    dst = edge_index[1]
    wcat1 = jnp.concatenate([Wroot1[None], Wr1], axis=0)
    wcat2 = jnp.concatenate([Wroot2[None], Wr2], axis=0)
    norm, gidx = _norm_kernel(src, dst, edge_type)
    t1 = _transform(x, wcat1)
    p1 = _agg_kernel(t1, dst, gidx, norm)
    t2 = _transform_fused(p1, t1, b1.reshape(1, D), wcat2)
    p2 = _agg_kernel(t2, dst, gidx, norm)
    return _combine(p2, t2, b2.reshape(1, D), relu=False)


# speedup vs baseline: 1.0203x; 1.0203x over previous
import functools

import jax
import jax.numpy as jnp
from jax import lax
from jax.experimental import pallas as pl
from jax.experimental.pallas import tpu as pltpu
from jax.experimental.pallas import tpu_sc as plsc

N = 10000
E = 320000
D = 128
R = 8
NC = 2
NS = 16
L = 16
EB = 80
GS = 4

CNT_PAD = 81920
E_PER_TILE_ALL = E // NS
E_PER_TILE_HALF = E // (NC * NS)

_mesh = plsc.VectorSubcoreMesh(
    core_axis_name="c", subcore_axis_name="s", num_cores=NC, num_subcores=NS)


def _zero_fill(ref, nwords):
    z = jnp.zeros((L,), dtype=ref.dtype)

    def body(i, _):
        ref[pl.ds(i * L, L)] = z
        return 0

    lax.fori_loop(0, nwords // L, body, 0)


def _grouped(n_blocks, group_fn):
    nfull = n_blocks // GS
    rem = n_blocks - nfull * GS

    def outer(go, _):
        group_fn(go * GS, GS)
        return 0

    lax.fori_loop(0, nfull, outer, 0)
    if rem:
        group_fn(nfull * GS, rem)


@functools.partial(
    pl.kernel,
    out_type=(
        jax.ShapeDtypeStruct((E,), jnp.float32),
        jax.ShapeDtypeStruct((E,), jnp.int32),
    ),
    mesh=_mesh,
    scratch_types=(
        pltpu.VMEM((CNT_PAD // NS,), jnp.float32),
        pltpu.VMEM((GS, EB), jnp.int32),
        pltpu.VMEM((GS, EB), jnp.int32),
        pltpu.VMEM((GS, EB), jnp.int32),
        pltpu.VMEM((GS, EB), jnp.int32),
        pltpu.VMEM((GS, EB), jnp.int32),
        pltpu.VMEM((GS, EB), jnp.float32),
        pltpu.VMEM((GS, EB), jnp.float32),
        pltpu.VMEM((EB,), jnp.float32),
        pltpu.VMEM_SHARED((CNT_PAD,), jnp.float32),
        (pltpu.SemaphoreType.DMA,) * GS,
        (pltpu.SemaphoreType.DMA,) * GS,
        (pltpu.SemaphoreType.DMA,) * GS,
    ),
)
def _norm_kernel(src_hbm, dst_hbm, et_hbm, norm_out, gidx_out,
                 zb, db, eb, sb, cb, gb, cntb, nb, ones, cnt_s,
                 sem_l, sem_cg, sem_st):
    c = lax.axis_index("c")
    s = lax.axis_index("s")

    _zero_fill(zb, CNT_PAD // NS)
    pltpu.sync_copy(zb, cnt_s.at[pl.ds(s * (CNT_PAD // NS), CNT_PAD // NS)])
    for k in range(EB // L):
        ones[pl.ds(k * L, L)] = jnp.ones((L,), jnp.float32)
    plsc.subcore_barrier()

    bbase = s * E_PER_TILE_ALL

    def count_group(blk0, count):
        base0 = bbase + blk0 * EB
        hs = []
        for j in range(count):
            base = base0 + j * EB
            hs.append((
                pltpu.async_copy(
                    dst_hbm.at[pl.ds(base, EB)], db.at[j], sem_l[j]),
                pltpu.async_copy(
                    et_hbm.at[pl.ds(base, EB)], eb.at[j], sem_l[j]),
            ))
        sh = []
        for j in range(count):
            for h in hs[j]:
                h.wait()
            for k in range(EB // L):
                ds = pl.ds(k * L, L)
                cb[j, ds] = db[j, ds] * R + eb[j, ds]
            sh.append(pltpu.async_copy(
                ones, cnt_s.at[cb.at[j]], sem_cg[j], add=True))
        for h in sh:
            h.wait()

    _grouped(E_PER_TILE_ALL // EB, count_group)
    plsc.subcore_barrier()

    cbase = c * (E // NC) + s * E_PER_TILE_HALF

    def norm_group(blk0, count):
        base0 = cbase + blk0 * EB
        hs = []
        for j in range(count):
            base = base0 + j * EB
            hs.append((
                pltpu.async_copy(
                    dst_hbm.at[pl.ds(base, EB)], db.at[j], sem_l[j]),
                pltpu.async_copy(
                    et_hbm.at[pl.ds(base, EB)], eb.at[j], sem_l[j]),
                pltpu.async_copy(
                    src_hbm.at[pl.ds(base, EB)], sb.at[j], sem_l[j]),
            ))
        cgh = []
        for j in range(count):
            for h in hs[j]:
                h.wait()
            for k in range(EB // L):
                ds = pl.ds(k * L, L)
                cb[j, ds] = db[j, ds] * R + eb[j, ds]
                gb[j, ds] = (eb[j, ds] + 1) * N + sb[j, ds]
            cgh.append(pltpu.async_copy(
                cnt_s.at[cb.at[j]], cntb.at[j], sem_cg[j]))
        sth = []
        for j in range(count):
            base = base0 + j * EB
            cgh[j].wait()
            for k in range(EB // L):
                ds = pl.ds(k * L, L)
                nb[j, ds] = 1.0 / jnp.maximum(cntb[j, ds], 1.0)
            sth.append(pltpu.async_copy(
                nb.at[j], norm_out.at[pl.ds(base, EB)], sem_st[j]))
            sth.append(pltpu.async_copy(
                gb.at[j], gidx_out.at[pl.ds(base, EB)], sem_st[j]))
        for h in sth:
            h.wait()

    _grouped(E_PER_TILE_HALF // EB, norm_group)


NP = 10240
N_PER_TILE = NP // NS


@functools.partial(
    pl.kernel,
    out_type=jax.ShapeDtypeStruct((NC, NP, D), jnp.float32),
    mesh=_mesh,
    scratch_types=(
        pltpu.VMEM((GS, EB), jnp.int32),
        pltpu.VMEM((GS, EB), jnp.int32),
        pltpu.VMEM((GS, EB + L), jnp.float32),
        pltpu.VMEM((GS, EB, D), jnp.float32),
        pltpu.VMEM_SHARED((NP, D), jnp.float32),
        (pltpu.SemaphoreType.DMA,) * GS,
        (pltpu.SemaphoreType.DMA,) * GS,
        (pltpu.SemaphoreType.DMA,) * GS,
    ),
)
def _agg_kernel(table_hbm, dst_hbm, gidx_hbm, norm_hbm, out,
                gb, db, nb, rows, acc_s, sem_l, sem_g, sem_s):
    c = lax.axis_index("c")
    s = lax.axis_index("s")
    ebase = c * (E // NC) + s * E_PER_TILE_HALF

    def zbody(i, _):
        for k in range(D // L):
            rows[0, i, pl.ds(k * L, L)] = jnp.zeros((L,), jnp.float32)
        return 0

    lax.fori_loop(0, EB, zbody, 0)
    for i in range(N_PER_TILE // EB):
        pltpu.sync_copy(
            rows.at[0], acc_s.at[pl.ds(s * N_PER_TILE + i * EB, EB), :])
    plsc.subcore_barrier()

    def edge_group(blk0, count):
        base0 = ebase + blk0 * EB
        hs = []
        for j in range(count):
            base = base0 + j * EB
            hs.append((
                pltpu.async_copy(
                    gidx_hbm.at[pl.ds(base, EB)], gb.at[j], sem_l[j]),
                pltpu.async_copy(
                    dst_hbm.at[pl.ds(base, EB)], db.at[j], sem_l[j]),
                pltpu.async_copy(
                    norm_hbm.at[pl.ds(base, EB)], nb.at[j, pl.ds(0, EB)],
                    sem_l[j]),
            ))
        gh = []
        for j in range(count):
            for h in hs[j]:
                h.wait()
            gh.append(pltpu.async_copy(
                table_hbm.at[gb.at[j]], rows.at[j], sem_g[j]))
        sh = []
        for j in range(count):
            gh[j].wait()

            def scale_chunk(m, _, j=j):
                nv16 = nb[j, pl.ds(m * L, L)]
                b0 = m * L
                for i in range(L):
                    nv = nv16[i]
                    for k in range(D // L):
                        ds = pl.ds(k * L, L)
                        rows[j, b0 + i, ds] = rows[j, b0 + i, ds] * nv
                return 0

            lax.fori_loop(0, EB // L, scale_chunk, 0)
            sh.append(pltpu.async_copy(
                rows.at[j], acc_s.at[db.at[j]], sem_s[j], add=True))
        for h in sh:
            h.wait()

    _grouped(E_PER_TILE_HALF // EB, edge_group)
    plsc.subcore_barrier()

    for i in range(N_PER_TILE // EB):
        rs = pl.ds(s * N_PER_TILE + i * EB, EB)
        pltpu.sync_copy(acc_s.at[rs, :], out.at[c, rs, :])


BN = 1000


def _transform(xin, wcat):
    nb = N // BN

    def body(x_ref, w_ref, o_ref):
        o_ref[...] = jnp.dot(x_ref[...], w_ref[0],
                             preferred_element_type=jnp.float32)

    return pl.pallas_call(
        body,
        grid=(R + 1, nb),
        in_specs=[
            pl.BlockSpec((BN, D), lambda g, n: (n, 0)),
            pl.BlockSpec((1, D, D), lambda g, n: (g, 0, 0)),
        ],
        out_specs=pl.BlockSpec((BN, D), lambda g, n: (g * (N // BN) + n, 0)),
        out_shape=jax.ShapeDtypeStruct(((R + 1) * N, D), jnp.float32),
    )(xin, wcat)


def _combine(partials, table, bias, relu):
    def body(p_ref, t_ref, b_ref, o_ref):
        v = p_ref[0] + p_ref[1] + t_ref[...] + b_ref[0]
        if relu:
            v = jnp.maximum(v, 0.0)
        o_ref[...] = v

    return pl.pallas_call(
        body,
        grid=(N // BN,),
        in_specs=[
            pl.BlockSpec((NC, BN, D), lambda n: (0, n, 0)),
            pl.BlockSpec((BN, D), lambda n: (n, 0)),
            pl.BlockSpec((1, D), lambda n: (0, 0)),
        ],
        out_specs=pl.BlockSpec((BN, D), lambda n: (n, 0)),
        out_shape=jax.ShapeDtypeStruct((N, D), jnp.float32),
    )(partials, table, bias)


def kernel(x, edge_index, edge_type, Wr1, Wroot1, b1, Wr2, Wroot2, b2):
    src = edge_index[0]
    dst = edge_index[1]
    wcat1 = jnp.concatenate([Wroot1[None], Wr1], axis=0)
    wcat2 = jnp.concatenate([Wroot2[None], Wr2], axis=0)
    norm, gidx = _norm_kernel(src, dst, edge_type)
    t1 = _transform(x, wcat1)
    p1 = _agg_kernel(t1, dst, gidx, norm)
    y1 = _combine(p1, t1, b1.reshape(1, D), relu=True)
    t2 = _transform(y1, wcat2)
    p2 = _agg_kernel(t2, dst, gidx, norm)
    return _combine(p2, t2, b2.reshape(1, D), relu=False)
